# Initial kernel scaffold; baseline (speedup 1.0000x reference)
import jax
import jax.numpy as jnp
from jax.experimental import pallas as pl


def kernel(in_feats, in_coords, weights):
    raise NotImplementedError("write your pallas kernel here")


# speedup vs baseline: 1.0529x; 1.0529x over previous
import functools

import jax
import jax.numpy as jnp
from jax.experimental import pallas as pl
from jax.experimental.pallas import tpu as pltpu

_OMEGA0 = 30.0
_A = 16.0
_N = 4
_LANES = 128


def _loe_kernel(coords_ref, feats_ref, wbig_ref, comp_ref, o_ref, *, G, Cout):
    fp = feats_ref[...]
    tr = fp.shape[0]
    y = jnp.dot(fp, wbig_ref[...], preferred_element_type=jnp.float32)

    cxy = jnp.floor(coords_ref[...] * _A).astype(jnp.int32) & 1
    t_of_lane = jax.lax.broadcasted_iota(jnp.int32, (tr, _LANES), 1) // Cout

    pieces = []
    for g in range(G):
        tile_g = (cxy[:, 2 * g:2 * g + 1] << 1) + cxy[:, 2 * g + 1:2 * g + 2]
        yg = y[:, _LANES * g:_LANES * (g + 1)]
        pieces.append(jnp.where(t_of_lane == tile_g, yg, 0.0))
    ym = jnp.concatenate(pieces, axis=-1)

    out = jnp.dot(ym, comp_ref[...], preferred_element_type=jnp.float32)
    o_ref[...] = jnp.sin(out)


def kernel(in_feats, in_coords, weights):
    B, Cin = in_feats.shape
    N, Cin_w, Cout = weights.shape
    G = _LANES // Cout

    feats_p = in_feats.reshape(B // G, G * Cin)
    coords_p = in_coords.reshape(B // G, G * 2)

    wall = (jnp.float32(_OMEGA0) * weights.astype(jnp.float32)
            ).transpose(1, 0, 2).reshape(Cin, N * Cout)
    wbig = jnp.kron(jnp.eye(G, dtype=jnp.float32), wall)

    rows = jnp.arange(G * N * Cout)
    cols = (rows // (N * Cout)) * Cout + rows % Cout
    comp = jnp.zeros((G * N * Cout, G * Cout), jnp.float32).at[rows, cols].set(1.0)

    tr = min(2048, B // G)
    n_steps = (B // G) // tr

    out_p = pl.pallas_call(
        functools.partial(_loe_kernel, G=G, Cout=Cout),
        out_shape=jax.ShapeDtypeStruct((B // G, G * Cout), jnp.float32),
        grid=(n_steps,),
        in_specs=[
            pl.BlockSpec((tr, G * 2), lambda i: (i, 0)),
            pl.BlockSpec((tr, G * Cin), lambda i: (i, 0)),
            pl.BlockSpec((G * Cin, G * N * Cout), lambda i: (0, 0)),
            pl.BlockSpec((G * N * Cout, G * Cout), lambda i: (0, 0)),
        ],
        out_specs=pl.BlockSpec((tr, G * Cout), lambda i: (i, 0)),
        compiler_params=pltpu.CompilerParams(
            dimension_semantics=("parallel",),
            vmem_limit_bytes=64 * 1024 * 1024),
    )(coords_p, feats_p, wbig, comp)

    return out_p.reshape(B, Cout), in_coords


# speedup vs baseline: 1.5685x; 1.4897x over previous
import functools

import jax
import jax.numpy as jnp
from jax.experimental import pallas as pl
from jax.experimental.pallas import tpu as pltpu

_OMEGA0 = 30.0
_A = 16.0
_LANES = 128

_INV_PI = 0.3183098861837907
_PI_HI = 3.140625
_PI_LO = 9.676535897932995e-04
_PI_LO2 = 2.3464020923e-10
_S1 = -0.16665840123183198
_S2 = 0.008314574141278546
_S3 = -0.0001856106694460715


def _cheap_sin(arg):
    kf = jnp.round(arg * _INV_PI)
    r = arg - kf * _PI_HI
    r = r - kf * _PI_LO
    r = r - kf * _PI_LO2
    k = kf.astype(jnp.int32)
    r2 = r * r
    p = (_S3 * r2 + _S2) * r2 + _S1
    s = r + r * (r2 * p)
    sbits = jax.lax.bitcast_convert_type(s, jnp.int32)
    sbits = sbits ^ ((k & 1) << 31)
    return jax.lax.bitcast_convert_type(sbits, jnp.float32)


def _loe_kernel(coords_ref, feats_ref, wall_ref, comp_ref, o_ref, *, Cout):
    x = feats_ref[...]
    rb = x.shape[0]
    y = jnp.dot(x, wall_ref[...], preferred_element_type=jnp.float32)

    cxy = jnp.floor(coords_ref[...] * _A).astype(jnp.int32) & 1
    tile = (cxy[:, 0:1] << 1) + cxy[:, 1:2]

    t_of_lane = jax.lax.broadcasted_iota(jnp.int32, (rb, _LANES), 1) // Cout
    ym = jnp.where(t_of_lane == tile, y, 0.0)

    s = _cheap_sin(ym)
    o_ref[...] = jnp.dot(s, comp_ref[...], preferred_element_type=jnp.float32)


def kernel(in_feats, in_coords, weights):
    B, Cin = in_feats.shape
    N, _, Cout = weights.shape

    wall = (jnp.float32(_OMEGA0) * weights.astype(jnp.float32)
            ).transpose(1, 0, 2).reshape(Cin, N * Cout)
    comp = jnp.tile(jnp.eye(Cout, dtype=jnp.float32), (N, 1))

    rb = min(4096, B)
    n_steps = B // rb

    out = pl.pallas_call(
        functools.partial(_loe_kernel, Cout=Cout),
        out_shape=jax.ShapeDtypeStruct((B, Cout), jnp.float32),
        grid=(n_steps,),
        in_specs=[
            pl.BlockSpec((rb, 2), lambda i: (i, 0)),
            pl.BlockSpec((rb, Cin), lambda i: (i, 0)),
            pl.BlockSpec((Cin, N * Cout), lambda i: (0, 0)),
            pl.BlockSpec((N * Cout, Cout), lambda i: (0, 0)),
        ],
        out_specs=pl.BlockSpec((rb, Cout), lambda i: (i, 0)),
        compiler_params=pltpu.CompilerParams(
            dimension_semantics=("parallel",),
            vmem_limit_bytes=64 * 1024 * 1024),
    )(in_coords, in_feats, wall, comp)

    return out, in_coords
